# baseline (speedup 1.0000x reference)
import functools

import jax
import jax.numpy as jnp
from jax import lax
from jax.experimental import pallas as pl
from jax.experimental.pallas import tpu as pltpu
from jax.experimental.pallas import tpu_sc as plsc

_D = 768
_E = 8
_S = 2048
_H = 3072
_FT = 768
_NF = _H // _FT
_TM = 256
_NPAD = 4096 + _E * _TM
_GM = _NPAD // _TM
_NEG = -1e30
_NTILE = 16
_TOKT = _S // _NTILE


def _gate_body(x_ref, wg_ref, bg_ref, g_ref, xbf_ref):
    xb = x_ref[...].astype(jnp.bfloat16)
    g = jax.lax.dot_general(
        xb, wg_ref[...].astype(jnp.bfloat16),
        (((1,), (0,)), ((), ())), preferred_element_type=jnp.float32)
    g = g + bg_ref[...]
    g_ref[...] = jnp.transpose(g[:, :_E])
    xbf_ref[...] = xb


def _gating(xs, Wg, bg):
    wg_pad = jnp.pad(Wg, ((0, 0), (0, 128 - _E)))
    bg_pad = jnp.pad(bg, (0, 128 - _E)).reshape(1, 128)
    return pl.pallas_call(
        _gate_body,
        grid=(_S // _TM,),
        in_specs=[
            pl.BlockSpec((_TM, _D), lambda m: (m, 0)),
            pl.BlockSpec((_D, 128), lambda m: (0, 0)),
            pl.BlockSpec((1, 128), lambda m: (0, 0)),
        ],
        out_specs=[pl.BlockSpec((_E, _TM), lambda m: (0, m)),
                   pl.BlockSpec((_TM, _D), lambda m: (m, 0))],
        out_shape=[jax.ShapeDtypeStruct((_E, _S), jnp.float32),
                   jax.ShapeDtypeStruct((_S, _D), jnp.bfloat16)],
    )(xs, wg_pad, bg_pad)


def _gat(x, idx):
    return x.at[idx].get(mode='promise_in_bounds')


def _route_body(g_hbm, src_tok_hbm, wsort_hbm, tile_e_hbm, pos0_hbm, pos1_hbm,
                g_v, idx_v, tokval_v, wval_v, cnt_v, allcnt_v, zi_v, zf_v,
                te_v, cnt_sh):
    cid = lax.axis_index("c")
    sid = lax.axis_index("s")

    @pl.when(cid == 0)
    def _():
        t0 = sid * _TOKT
        for e in range(_E):
            pltpu.sync_copy(g_hbm.at[e, pl.ds(t0, _TOKT)], g_v.at[e])

        zslice = _NPAD // _NTILE
        z16i = jnp.zeros((16,), jnp.int32)
        z16f = jnp.zeros((16,), jnp.float32)
        for i in range(zslice // 16):
            zi_v[pl.ds(i * 16, 16)] = z16i
            zf_v[pl.ds(i * 16, 16)] = z16f
        pltpu.sync_copy(zi_v, src_tok_hbm.at[pl.ds(sid * zslice, zslice)])
        pltpu.sync_copy(zf_v, wsort_hbm.at[pl.ds(sid * zslice, zslice)])

        iota = lax.iota(jnp.int32, 16)

        def shdown(x, k):
            g = _gat(x, jnp.maximum(iota - k, 0))
            return jnp.where(iota >= k, g, 0)

        def prefix_excl(x):
            s = x
            for k in (1, 2, 4, 8):
                s = s + shdown(s, k)
            return s - x

        ngroups = _TOKT // 16
        e_sl, w_sl = [[], []], [[], []]
        for v in range(ngroups):
            gs = [g_v[e, pl.ds(v * 16, 16)] for e in range(_E)]
            m1 = gs[0]
            i1 = jnp.zeros((16,), jnp.int32)
            for e in range(1, _E):
                better = gs[e] > m1
                m1 = jnp.where(better, gs[e], m1)
                i1 = jnp.where(better, e, i1)
            m2 = jnp.full((16,), _NEG, jnp.float32)
            i2 = jnp.zeros((16,), jnp.int32)
            for e in range(_E):
                cand = jnp.where(i1 == e, _NEG, gs[e])
                better = cand > m2
                m2 = jnp.where(better, cand, m2)
                i2 = jnp.where(better, e, i2)
            p1 = 1.0 / (1.0 + jnp.exp(m2 - m1))
            e_sl[0].append(i1)
            e_sl[1].append(i2)
            w_sl[0].append(p1)
            w_sl[1].append(1.0 - p1)

        full15 = jnp.full((16,), 15, jnp.int32)
        run_lo = jnp.zeros((16,), jnp.int32)
        run_hi = jnp.zeros((16,), jnp.int32)
        ranks = []
        for ev in e_sl[0] + e_sl[1]:
            sh_lo = jnp.minimum(8 * ev, 24)
            sh_hi = 8 * jnp.clip(ev - 4, 0, 3)
            enc_lo = jnp.where(ev < 4, jnp.left_shift(1, sh_lo), 0)
            enc_hi = jnp.where(ev >= 4, jnp.left_shift(1, sh_hi), 0)
            pe_lo = prefix_excl(enc_lo)
            pe_hi = prefix_excl(enc_hi)
            r = jnp.where(
                ev < 4,
                jnp.right_shift(run_lo + pe_lo, sh_lo) & 255,
                jnp.right_shift(run_hi + pe_hi, sh_hi) & 255)
            ranks.append(r)
            run_lo = run_lo + _gat(pe_lo + enc_lo, full15)
            run_hi = run_hi + _gat(pe_hi + enc_hi, full15)

        cvec = jnp.zeros((16,), jnp.int32)
        for e in range(_E):
            run = run_lo if e < 4 else run_hi
            cvec = cvec + jnp.where(
                iota == e, jnp.right_shift(run, 8 * (e % 4)) & 255, 0)
        cnt_v[...] = cvec
        pltpu.sync_copy(cnt_v, cnt_sh.at[pl.ds(sid * 16, 16)])
        plsc.subcore_barrier()
        pltpu.sync_copy(cnt_sh, allcnt_v)

        prior = jnp.zeros((16,), jnp.int32)
        total = jnp.zeros((16,), jnp.int32)
        sidv = iota * 0 + sid
        for t in range(_NTILE):
            ct = allcnt_v[pl.ds(t * 16, 16)]
            total = total + ct
            prior = prior + ct * jnp.clip(sidv - t, 0, 1)
        padded = ((total + (_TM - 1)) >> 8) << 8
        seg = prefix_excl(padded)
        base = seg + prior

        for k, ev in enumerate(e_sl[0] + e_sl[1]):
            pos = ranks[k] + _gat(base, ev)
            slot, v = divmod(k, ngroups)
            idx_v[slot, pl.ds(v * 16, 16)] = pos
            tokval_v[slot, pl.ds(v * 16, 16)] = t0 + v * 16 + iota
            wval_v[slot, pl.ds(v * 16, 16)] = w_sl[slot][v]

        for j in range(2):
            pltpu.sync_copy(tokval_v.at[j], src_tok_hbm.at[idx_v.at[j]])
            pltpu.sync_copy(wval_v.at[j], wsort_hbm.at[idx_v.at[j]])
        pltpu.sync_copy(idx_v.at[0], pos0_hbm.at[pl.ds(t0, _TOKT)])
        pltpu.sync_copy(idx_v.at[1], pos1_hbm.at[pl.ds(t0, _TOKT)])

        @pl.when(sid == 0)
        def _tiles():
            seg7 = _gat(seg, jnp.full((16,), 7, jnp.int32))
            pad7 = _gat(padded, jnp.full((16,), 7, jnp.int32))
            tot_pad = seg7 + pad7
            for half in range(2):
                mv = (iota + half * 16) * _TM
                te = jnp.zeros((16,), jnp.int32)
                for e in range(_E):
                    se = _gat(seg, jnp.full((16,), e, jnp.int32))
                    pe = _gat(padded, jnp.full((16,), e, jnp.int32))
                    inseg = jnp.logical_and(mv >= se, mv < se + pe)
                    te = te + jnp.where(inseg, e, 0)
                te = jnp.where(mv >= tot_pad, _E - 1, te)
                te_v[pl.ds(half * 16, 16)] = te
            pltpu.sync_copy(te_v, tile_e_hbm)


def _routing(gates):
    mesh = plsc.VectorSubcoreMesh(core_axis_name="c", subcore_axis_name="s")
    k = functools.partial(
        pl.kernel, mesh=mesh,
        out_type=[
            jax.ShapeDtypeStruct((_NPAD,), jnp.int32),
            jax.ShapeDtypeStruct((_NPAD,), jnp.float32),
            jax.ShapeDtypeStruct((32,), jnp.int32),
            jax.ShapeDtypeStruct((_S,), jnp.int32),
            jax.ShapeDtypeStruct((_S,), jnp.int32),
        ],
        scratch_types=[
            pltpu.VMEM((_E, _TOKT), jnp.float32),
            pltpu.VMEM((2, _TOKT), jnp.int32),
            pltpu.VMEM((2, _TOKT), jnp.int32),
            pltpu.VMEM((2, _TOKT), jnp.float32),
            pltpu.VMEM((16,), jnp.int32),
            pltpu.VMEM((_NTILE * 16,), jnp.int32),
            pltpu.VMEM((_NPAD // _NTILE,), jnp.int32),
            pltpu.VMEM((_NPAD // _NTILE,), jnp.float32),
            pltpu.VMEM((32,), jnp.int32),
            pltpu.VMEM_SHARED((_NTILE * 16,), jnp.int32),
        ],
    )(_route_body)
    return k(gates)


def _xgather_body(src_tok_hbm, x_hbm, xsort_hbm, cidx_v, rows0_v, rows1_v,
                  sem0, sem1, semw0, semw1):
    wid = lax.axis_index("s") * 2 + lax.axis_index("c")
    rows = _NPAD // 32
    half = rows // 2
    base = wid * rows
    pltpu.sync_copy(src_tok_hbm.at[pl.ds(base, half)], cidx_v.at[0])
    pltpu.sync_copy(src_tok_hbm.at[pl.ds(base + half, half)], cidx_v.at[1])
    g0 = pltpu.async_copy(x_hbm.at[cidx_v.at[0]], rows0_v, sem0)
    g1 = pltpu.async_copy(x_hbm.at[cidx_v.at[1]], rows1_v, sem1)
    g0.wait()
    w0 = pltpu.async_copy(rows0_v, xsort_hbm.at[pl.ds(base, half)], semw0)
    g1.wait()
    w1 = pltpu.async_copy(rows1_v, xsort_hbm.at[pl.ds(base + half, half)],
                          semw1)
    w0.wait()
    w1.wait()


def _xgather(src_tok, xi):
    mesh = plsc.VectorSubcoreMesh(core_axis_name="c", subcore_axis_name="s")
    k = functools.partial(
        pl.kernel, mesh=mesh,
        out_type=jax.ShapeDtypeStruct((_NPAD, _D // 2), jnp.int32),
        scratch_types=[
            pltpu.VMEM((2, _NPAD // 64), jnp.int32),
            pltpu.VMEM((_NPAD // 64, _D // 2), jnp.int32),
            pltpu.VMEM((_NPAD // 64, _D // 2), jnp.int32),
            pltpu.SemaphoreType.DMA,
            pltpu.SemaphoreType.DMA,
            pltpu.SemaphoreType.DMA,
            pltpu.SemaphoreType.DMA,
        ],
    )(_xgather_body)
    return k(src_tok, xi)


def _ffn_body(te_ref, x_ref, w1_ref, b1_ref, w2_ref, b2_ref, ws_ref, y_ref,
              acc_ref):
    f = pl.program_id(0)
    m = pl.program_id(1)
    sl = pl.ds(m * _TM, _TM)
    xm = x_ref[sl, :]
    h = jax.lax.dot_general(
        xm, w1_ref[0].astype(jnp.bfloat16),
        (((1,), (0,)), ((), ())), preferred_element_type=jnp.float32)
    h = h + b1_ref[0]
    h = 0.5 * h * (1.0 + jax.lax.erf(h * 0.7071067811865476))
    part = jax.lax.dot_general(
        h.astype(jnp.bfloat16), w2_ref[0].astype(jnp.bfloat16),
        (((1,), (0,)), ((), ())), preferred_element_type=jnp.float32)

    @pl.when(f == 0)
    def _():
        acc_ref[sl, :] = part + b2_ref[0]

    @pl.when(jnp.logical_and(f > 0, f < _NF - 1))
    def _():
        acc_ref[sl, :] += part

    @pl.when(f == _NF - 1)
    def _():
        y_ref[sl, :] = (acc_ref[sl, :] + part) * ws_ref[sl, :]


def _ffn(tile_e, x_sorted, W1, b1, W2, b2, wsort):
    grid_spec = pltpu.PrefetchScalarGridSpec(
        num_scalar_prefetch=1,
        grid=(_NF, _GM),
        in_specs=[
            pl.BlockSpec((_NPAD, _D), lambda f, m, te: (0, 0)),
            pl.BlockSpec((1, _D, _FT), lambda f, m, te: (te[m], 0, f)),
            pl.BlockSpec((1, 1, _FT), lambda f, m, te: (te[m], 0, f)),
            pl.BlockSpec((1, _FT, _D), lambda f, m, te: (te[m], f, 0)),
            pl.BlockSpec((1, 1, _D), lambda f, m, te: (te[m], 0, 0)),
            pl.BlockSpec((_NPAD, 1), lambda f, m, te: (0, 0)),
        ],
        out_specs=pl.BlockSpec((_NPAD, _D), lambda f, m, te: (0, 0)),
        scratch_shapes=[pltpu.VMEM((_NPAD, _D), jnp.float32)],
    )
    return pl.pallas_call(
        _ffn_body,
        grid_spec=grid_spec,
        out_shape=jax.ShapeDtypeStruct((_NPAD, _D), jnp.float32),
    )(tile_e, x_sorted, W1, b1.reshape(_E, 1, _H), W2,
      b2.reshape(_E, 1, _D), wsort.reshape(_NPAD, 1))


def _combine_body(pos0_hbm, pos1_hbm, y_hbm, out_hbm,
                  p0_v, p1_v, ra_v, rb_v, sem):
    wid = lax.axis_index("s") * 2 + lax.axis_index("c")
    toks = _S // 32
    ck = 16
    t0 = wid * toks
    pltpu.sync_copy(pos0_hbm.at[pl.ds(t0, toks)], p0_v)
    pltpu.sync_copy(pos1_hbm.at[pl.ds(t0, toks)], p1_v)

    def chunk(j, carry):
        pltpu.async_copy(y_hbm.at[p0_v.at[pl.ds(j * ck, ck)]], ra_v,
                         sem).wait()
        pltpu.async_copy(y_hbm.at[p1_v.at[pl.ds(j * ck, ck)]], rb_v,
                         sem).wait()
        for r in range(ck):
            for c in range(_D // 16):
                cs = pl.ds(c * 16, 16)
                rb_v[r, cs] = ra_v[r, cs] + rb_v[r, cs]
        pltpu.sync_copy(rb_v, out_hbm.at[pl.ds(t0 + j * ck, ck)])
        return carry

    lax.fori_loop(0, toks // ck, chunk, 0)


def _combine(pos0, pos1, y_sorted):
    mesh = plsc.VectorSubcoreMesh(core_axis_name="c", subcore_axis_name="s")
    k = functools.partial(
        pl.kernel, mesh=mesh,
        out_type=jax.ShapeDtypeStruct((_S, _D), jnp.float32),
        scratch_types=[
            pltpu.VMEM((_S // 32,), jnp.int32),
            pltpu.VMEM((_S // 32,), jnp.int32),
            pltpu.VMEM((16, _D), jnp.float32),
            pltpu.VMEM((16, _D), jnp.float32),
            pltpu.SemaphoreType.DMA,
        ],
    )(_combine_body)
    return k(pos0, pos1, y_sorted)


def kernel(x, Wg, bg, W1, b1, W2, b2):
    xs = x.reshape(_S, _D)
    gates, xbf = _gating(xs, Wg, bg)
    src_tok, wsort, tile_e, pos0, pos1 = _routing(gates)
    xi = jax.lax.bitcast_convert_type(
        xbf.reshape(_S, _D // 2, 2), jnp.int32)
    xsi = _xgather(src_tok, xi)
    x_sorted = jax.lax.bitcast_convert_type(
        xsi, jnp.bfloat16).reshape(_NPAD, _D)
    y_sorted = _ffn(tile_e, x_sorted, W1, b1, W2, b2, wsort)
    out = _combine(pos0, pos1, y_sorted)
    return out.reshape(1, _S, _D)


# speedup vs baseline: 1.3066x; 1.3066x over previous
import functools

import jax
import jax.numpy as jnp
from jax import lax
from jax.experimental import pallas as pl
from jax.experimental.pallas import tpu as pltpu
from jax.experimental.pallas import tpu_sc as plsc

_D = 768
_E = 8
_S = 2048
_H = 3072
_FT = 768
_NF = _H // _FT
_TM = 256
_NPAD = 4096 + _E * _TM
_GM = _NPAD // _TM
_NEG = -1e30
_NTILE = 16
_TOKT = _S // _NTILE


def _gate_body(x_ref, wg_ref, bg_ref, g_ref):
    xb = x_ref[...].astype(jnp.bfloat16)
    g = jax.lax.dot_general(
        xb, wg_ref[...].astype(jnp.bfloat16),
        (((1,), (0,)), ((), ())), preferred_element_type=jnp.float32)
    g = g + bg_ref[...]
    g_ref[...] = jnp.transpose(g[:, :_E])


def _gating(xs, Wg, bg):
    wg_pad = jnp.pad(Wg, ((0, 0), (0, 128 - _E)))
    bg_pad = jnp.pad(bg, (0, 128 - _E)).reshape(1, 128)
    return pl.pallas_call(
        _gate_body,
        grid=(_S // _TM,),
        in_specs=[
            pl.BlockSpec((_TM, _D), lambda m: (m, 0)),
            pl.BlockSpec((_D, 128), lambda m: (0, 0)),
            pl.BlockSpec((1, 128), lambda m: (0, 0)),
        ],
        out_specs=pl.BlockSpec((_E, _TM), lambda m: (0, m)),
        out_shape=jax.ShapeDtypeStruct((_E, _S), jnp.float32),
    )(xs, wg_pad, bg_pad)


def _gat(x, idx):
    return x.at[idx].get(mode='promise_in_bounds')


def _route_body(g_hbm, src_tok_hbm, wsort_hbm, tile_e_hbm, pos0_hbm, pos1_hbm,
                g_v, idx_v, tokval_v, wval_v, cnt_v, allcnt_v, zi_v, zf_v,
                te_v, cnt_sh):
    cid = lax.axis_index("c")
    sid = lax.axis_index("s")

    @pl.when(cid == 0)
    def _():
        t0 = sid * _TOKT
        for e in range(_E):
            pltpu.sync_copy(g_hbm.at[e, pl.ds(t0, _TOKT)], g_v.at[e])

        zslice = _NPAD // _NTILE
        z16i = jnp.zeros((16,), jnp.int32)
        z16f = jnp.zeros((16,), jnp.float32)
        for i in range(zslice // 16):
            zi_v[pl.ds(i * 16, 16)] = z16i
            zf_v[pl.ds(i * 16, 16)] = z16f
        pltpu.sync_copy(zi_v, src_tok_hbm.at[pl.ds(sid * zslice, zslice)])
        pltpu.sync_copy(zf_v, wsort_hbm.at[pl.ds(sid * zslice, zslice)])

        iota = lax.iota(jnp.int32, 16)

        def shdown(x, k):
            g = _gat(x, jnp.maximum(iota - k, 0))
            return jnp.where(iota >= k, g, 0)

        def prefix_excl(x):
            s = x
            for k in (1, 2, 4, 8):
                s = s + shdown(s, k)
            return s - x

        ngroups = _TOKT // 16
        e_sl, w_sl = [[], []], [[], []]
        for v in range(ngroups):
            gs = [g_v[e, pl.ds(v * 16, 16)] for e in range(_E)]
            m1 = gs[0]
            i1 = jnp.zeros((16,), jnp.int32)
            for e in range(1, _E):
                better = gs[e] > m1
                m1 = jnp.where(better, gs[e], m1)
                i1 = jnp.where(better, e, i1)
            m2 = jnp.full((16,), _NEG, jnp.float32)
            i2 = jnp.zeros((16,), jnp.int32)
            for e in range(_E):
                cand = jnp.where(i1 == e, _NEG, gs[e])
                better = cand > m2
                m2 = jnp.where(better, cand, m2)
                i2 = jnp.where(better, e, i2)
            p1 = 1.0 / (1.0 + jnp.exp(m2 - m1))
            e_sl[0].append(i1)
            e_sl[1].append(i2)
            w_sl[0].append(p1)
            w_sl[1].append(1.0 - p1)

        full15 = jnp.full((16,), 15, jnp.int32)
        run_lo = jnp.zeros((16,), jnp.int32)
        run_hi = jnp.zeros((16,), jnp.int32)
        ranks = []
        for ev in e_sl[0] + e_sl[1]:
            sh_lo = jnp.minimum(8 * ev, 24)
            sh_hi = 8 * jnp.clip(ev - 4, 0, 3)
            enc_lo = jnp.where(ev < 4, jnp.left_shift(1, sh_lo), 0)
            enc_hi = jnp.where(ev >= 4, jnp.left_shift(1, sh_hi), 0)
            pe_lo = prefix_excl(enc_lo)
            pe_hi = prefix_excl(enc_hi)
            r = jnp.where(
                ev < 4,
                jnp.right_shift(run_lo + pe_lo, sh_lo) & 255,
                jnp.right_shift(run_hi + pe_hi, sh_hi) & 255)
            ranks.append(r)
            run_lo = run_lo + _gat(pe_lo + enc_lo, full15)
            run_hi = run_hi + _gat(pe_hi + enc_hi, full15)

        cvec = jnp.zeros((16,), jnp.int32)
        for e in range(_E):
            run = run_lo if e < 4 else run_hi
            cvec = cvec + jnp.where(
                iota == e, jnp.right_shift(run, 8 * (e % 4)) & 255, 0)
        cnt_v[...] = cvec
        pltpu.sync_copy(cnt_v, cnt_sh.at[pl.ds(sid * 16, 16)])
        plsc.subcore_barrier()
        pltpu.sync_copy(cnt_sh, allcnt_v)

        prior = jnp.zeros((16,), jnp.int32)
        total = jnp.zeros((16,), jnp.int32)
        sidv = iota * 0 + sid
        for t in range(_NTILE):
            ct = allcnt_v[pl.ds(t * 16, 16)]
            total = total + ct
            prior = prior + ct * jnp.clip(sidv - t, 0, 1)
        padded = ((total + (_TM - 1)) >> 8) << 8
        seg = prefix_excl(padded)
        base = seg + prior

        for k, ev in enumerate(e_sl[0] + e_sl[1]):
            pos = ranks[k] + _gat(base, ev)
            slot, v = divmod(k, ngroups)
            idx_v[slot, pl.ds(v * 16, 16)] = pos
            tokval_v[slot, pl.ds(v * 16, 16)] = t0 + v * 16 + iota
            wval_v[slot, pl.ds(v * 16, 16)] = w_sl[slot][v]

        for j in range(2):
            pltpu.sync_copy(tokval_v.at[j], src_tok_hbm.at[idx_v.at[j]])
            pltpu.sync_copy(wval_v.at[j], wsort_hbm.at[idx_v.at[j]])
        pltpu.sync_copy(idx_v.at[0], pos0_hbm.at[pl.ds(t0, _TOKT)])
        pltpu.sync_copy(idx_v.at[1], pos1_hbm.at[pl.ds(t0, _TOKT)])

        @pl.when(sid == 0)
        def _tiles():
            seg7 = _gat(seg, jnp.full((16,), 7, jnp.int32))
            pad7 = _gat(padded, jnp.full((16,), 7, jnp.int32))
            tot_pad = seg7 + pad7
            for half in range(2):
                mv = (iota + half * 16) * _TM
                te = jnp.zeros((16,), jnp.int32)
                for e in range(_E):
                    se = _gat(seg, jnp.full((16,), e, jnp.int32))
                    pe = _gat(padded, jnp.full((16,), e, jnp.int32))
                    inseg = jnp.logical_and(mv >= se, mv < se + pe)
                    te = te + jnp.where(inseg, e, 0)
                te = jnp.where(mv >= tot_pad, _E - 1, te)
                te_v[pl.ds(half * 16, 16)] = te
            pltpu.sync_copy(te_v, tile_e_hbm)


def _routing(gates):
    mesh = plsc.VectorSubcoreMesh(core_axis_name="c", subcore_axis_name="s")
    k = functools.partial(
        pl.kernel, mesh=mesh,
        out_type=[
            jax.ShapeDtypeStruct((_NPAD,), jnp.int32),
            jax.ShapeDtypeStruct((_NPAD,), jnp.float32),
            jax.ShapeDtypeStruct((32,), jnp.int32),
            jax.ShapeDtypeStruct((_S,), jnp.int32),
            jax.ShapeDtypeStruct((_S,), jnp.int32),
        ],
        scratch_types=[
            pltpu.VMEM((_E, _TOKT), jnp.float32),
            pltpu.VMEM((2, _TOKT), jnp.int32),
            pltpu.VMEM((2, _TOKT), jnp.int32),
            pltpu.VMEM((2, _TOKT), jnp.float32),
            pltpu.VMEM((16,), jnp.int32),
            pltpu.VMEM((_NTILE * 16,), jnp.int32),
            pltpu.VMEM((_NPAD // _NTILE,), jnp.int32),
            pltpu.VMEM((_NPAD // _NTILE,), jnp.float32),
            pltpu.VMEM((32,), jnp.int32),
            pltpu.VMEM_SHARED((_NTILE * 16,), jnp.int32),
        ],
    )(_route_body)
    return k(gates)


def _xgather_body(src_tok_hbm, x_hbm, xsort_hbm, cidx_v, rows0_v, rows1_v,
                  semg0, semg1, semw0, semw1):
    wid = lax.axis_index("s") * 2 + lax.axis_index("c")
    rows = _NPAD // 32
    ck = rows // 4
    base = wid * rows
    pltpu.sync_copy(src_tok_hbm.at[pl.ds(base, rows)], cidx_v)
    bufs = (rows0_v, rows1_v)
    gsems = (semg0, semg1)
    wsems = (semw0, semw1)
    g = [None] * 4
    w = [None] * 4
    g[0] = pltpu.async_copy(x_hbm.at[cidx_v.at[pl.ds(0, ck)]], rows0_v, semg0)
    for c in range(4):
        g[c].wait()
        w[c] = pltpu.async_copy(
            bufs[c % 2], xsort_hbm.at[pl.ds(base + c * ck, ck)],
            wsems[c % 2])
        if c + 1 < 4:
            if c >= 1:
                w[c - 1].wait()
            g[c + 1] = pltpu.async_copy(
                x_hbm.at[cidx_v.at[pl.ds((c + 1) * ck, ck)]],
                bufs[(c + 1) % 2], gsems[(c + 1) % 2])
    w[2].wait()
    w[3].wait()


def _xgather(src_tok, xs):
    mesh = plsc.VectorSubcoreMesh(core_axis_name="c", subcore_axis_name="s")
    k = functools.partial(
        pl.kernel, mesh=mesh,
        out_type=jax.ShapeDtypeStruct((_NPAD, _D), jnp.float32),
        scratch_types=[
            pltpu.VMEM((_NPAD // 32,), jnp.int32),
            pltpu.VMEM((_NPAD // 128, _D), jnp.float32),
            pltpu.VMEM((_NPAD // 128, _D), jnp.float32),
            pltpu.SemaphoreType.DMA,
            pltpu.SemaphoreType.DMA,
            pltpu.SemaphoreType.DMA,
            pltpu.SemaphoreType.DMA,
        ],
    )(_xgather_body)
    return k(src_tok, xs)


def _ffn_body(te_ref, x_ref, w1_ref, b1_ref, w2_ref, b2_ref, ws_ref, y_ref):
    f = pl.program_id(0)
    m = pl.program_id(1)
    sl = pl.ds(m * _TM, _TM)
    xm = x_ref[sl, :].astype(jnp.bfloat16)
    h = jax.lax.dot_general(
        xm, w1_ref[0].astype(jnp.bfloat16),
        (((1,), (0,)), ((), ())), preferred_element_type=jnp.float32)
    h = h + b1_ref[0]
    h = 0.5 * h * (1.0 + jax.lax.erf(h * 0.7071067811865476))
    part = jax.lax.dot_general(
        h.astype(jnp.bfloat16), w2_ref[0].astype(jnp.bfloat16),
        (((1,), (0,)), ((), ())), preferred_element_type=jnp.float32)

    @pl.when(f == 0)
    def _():
        y_ref[sl, :] = part + b2_ref[0]

    @pl.when(jnp.logical_and(f > 0, f < _NF - 1))
    def _():
        y_ref[sl, :] += part

    @pl.when(f == _NF - 1)
    def _():
        y_ref[sl, :] = (y_ref[sl, :] + part) * ws_ref[sl, :]


def _ffn(tile_e, x_sorted, W1, b1, W2, b2, wsort):
    grid_spec = pltpu.PrefetchScalarGridSpec(
        num_scalar_prefetch=1,
        grid=(_NF, _GM),
        in_specs=[
            pl.BlockSpec((_NPAD, _D), lambda f, m, te: (0, 0)),
            pl.BlockSpec((1, _D, _FT), lambda f, m, te: (te[m], 0, f)),
            pl.BlockSpec((1, 1, _FT), lambda f, m, te: (te[m], 0, f)),
            pl.BlockSpec((1, _FT, _D), lambda f, m, te: (te[m], f, 0)),
            pl.BlockSpec((1, 1, _D), lambda f, m, te: (te[m], 0, 0)),
            pl.BlockSpec((_NPAD, 1), lambda f, m, te: (0, 0)),
        ],
        out_specs=pl.BlockSpec((_NPAD, _D), lambda f, m, te: (0, 0)),
    )
    return pl.pallas_call(
        _ffn_body,
        grid_spec=grid_spec,
        out_shape=jax.ShapeDtypeStruct((_NPAD, _D), jnp.float32),
    )(tile_e, x_sorted, W1, b1.reshape(_E, 1, _H), W2,
      b2.reshape(_E, 1, _D), wsort.reshape(_NPAD, 1))


def _combine_body(pos0_hbm, pos1_hbm, y_hbm, out_hbm,
                  p0_v, p1_v, ra_v, rb_v, sem):
    wid = lax.axis_index("s") * 2 + lax.axis_index("c")
    toks = _S // 32
    ck = 16
    t0 = wid * toks
    pltpu.sync_copy(pos0_hbm.at[pl.ds(t0, toks)], p0_v)
    pltpu.sync_copy(pos1_hbm.at[pl.ds(t0, toks)], p1_v)

    def chunk(j, carry):
        pltpu.async_copy(y_hbm.at[p0_v.at[pl.ds(j * ck, ck)]], ra_v,
                         sem).wait()
        pltpu.async_copy(y_hbm.at[p1_v.at[pl.ds(j * ck, ck)]], rb_v,
                         sem).wait()
        for r in range(ck):
            for c in range(_D // 16):
                cs = pl.ds(c * 16, 16)
                rb_v[r, cs] = ra_v[r, cs] + rb_v[r, cs]
        pltpu.sync_copy(rb_v, out_hbm.at[pl.ds(t0 + j * ck, ck)])
        return carry

    lax.fori_loop(0, toks // ck, chunk, 0)


def _combine(pos0, pos1, y_sorted):
    mesh = plsc.VectorSubcoreMesh(core_axis_name="c", subcore_axis_name="s")
    k = functools.partial(
        pl.kernel, mesh=mesh,
        out_type=jax.ShapeDtypeStruct((_S, _D), jnp.float32),
        scratch_types=[
            pltpu.VMEM((_S // 32,), jnp.int32),
            pltpu.VMEM((_S // 32,), jnp.int32),
            pltpu.VMEM((16, _D), jnp.float32),
            pltpu.VMEM((16, _D), jnp.float32),
            pltpu.SemaphoreType.DMA,
        ],
    )(_combine_body)
    return k(pos0, pos1, y_sorted)


def kernel(x, Wg, bg, W1, b1, W2, b2):
    xs = x.reshape(_S, _D)
    gates = _gating(xs, Wg, bg)
    src_tok, wsort, tile_e, pos0, pos1 = _routing(gates)
    x_sorted = _xgather(src_tok, xs)
    y_sorted = _ffn(tile_e, x_sorted, W1, b1, W2, b2, wsort)
    out = _combine(pos0, pos1, y_sorted)
    return out.reshape(1, _S, _D)


# speedup vs baseline: 1.7531x; 1.3417x over previous
import functools

import jax
import jax.numpy as jnp
from jax import lax
from jax.experimental import pallas as pl
from jax.experimental.pallas import tpu as pltpu
from jax.experimental.pallas import tpu_sc as plsc

_D = 768
_E = 8
_S = 2048
_H = 3072
_FT = 768
_NF = _H // _FT
_TM = 256
_NPAD = 4096 + _E * _TM
_GM = _NPAD // _TM
_NEG = -1e30
_NTILE = 16
_TOKT = _S // _NTILE


def _gate_body(x_ref, wg_ref, bg_ref, g_ref):
    xb = x_ref[...].astype(jnp.bfloat16)
    g = jax.lax.dot_general(
        xb, wg_ref[...].astype(jnp.bfloat16),
        (((1,), (0,)), ((), ())), preferred_element_type=jnp.float32)
    g = g + bg_ref[...]
    g_ref[...] = jnp.transpose(g[:, :_E])


def _gating(xs, Wg, bg):
    wg_pad = jnp.pad(Wg, ((0, 0), (0, 128 - _E)))
    bg_pad = jnp.pad(bg, (0, 128 - _E)).reshape(1, 128)
    return pl.pallas_call(
        _gate_body,
        grid=(_S // _TM,),
        in_specs=[
            pl.BlockSpec((_TM, _D), lambda m: (m, 0)),
            pl.BlockSpec((_D, 128), lambda m: (0, 0)),
            pl.BlockSpec((1, 128), lambda m: (0, 0)),
        ],
        out_specs=pl.BlockSpec((_E, _TM), lambda m: (0, m)),
        out_shape=jax.ShapeDtypeStruct((_E, _S), jnp.float32),
    )(xs, wg_pad, bg_pad)


def _gat(x, idx):
    return x.at[idx].get(mode='promise_in_bounds')


def _route_body(g_hbm, src_tok_hbm, wsort_hbm, tile_e_hbm, pos0_hbm, pos1_hbm,
                g_v, idx_v, tokval_v, wval_v, cnt_v, allcnt_v, zi_v, zf_v,
                te_v, cnt_sh):
    cid = lax.axis_index("c")
    sid = lax.axis_index("s")

    @pl.when(cid == 0)
    def _():
        t0 = sid * _TOKT
        for e in range(_E):
            pltpu.sync_copy(g_hbm.at[e, pl.ds(t0, _TOKT)], g_v.at[e])

        zslice = _NPAD // _NTILE
        z16i = jnp.zeros((16,), jnp.int32)
        z16f = jnp.zeros((16,), jnp.float32)
        for i in range(zslice // 16):
            zi_v[pl.ds(i * 16, 16)] = z16i
            zf_v[pl.ds(i * 16, 16)] = z16f
        pltpu.sync_copy(zi_v, src_tok_hbm.at[pl.ds(sid * zslice, zslice)])
        pltpu.sync_copy(zf_v, wsort_hbm.at[pl.ds(sid * zslice, zslice)])

        iota = lax.iota(jnp.int32, 16)

        def shdown(x, k):
            g = _gat(x, jnp.maximum(iota - k, 0))
            return jnp.where(iota >= k, g, 0)

        def prefix_excl(x):
            s = x
            for k in (1, 2, 4, 8):
                s = s + shdown(s, k)
            return s - x

        ngroups = _TOKT // 16
        e_sl, w_sl = [[], []], [[], []]
        for v in range(ngroups):
            gs = [g_v[e, pl.ds(v * 16, 16)] for e in range(_E)]
            m1 = gs[0]
            i1 = jnp.zeros((16,), jnp.int32)
            for e in range(1, _E):
                better = gs[e] > m1
                m1 = jnp.where(better, gs[e], m1)
                i1 = jnp.where(better, e, i1)
            m2 = jnp.full((16,), _NEG, jnp.float32)
            i2 = jnp.zeros((16,), jnp.int32)
            for e in range(_E):
                cand = jnp.where(i1 == e, _NEG, gs[e])
                better = cand > m2
                m2 = jnp.where(better, cand, m2)
                i2 = jnp.where(better, e, i2)
            p1 = 1.0 / (1.0 + jnp.exp(m2 - m1))
            e_sl[0].append(i1)
            e_sl[1].append(i2)
            w_sl[0].append(p1)
            w_sl[1].append(1.0 - p1)

        full15 = jnp.full((16,), 15, jnp.int32)
        run_lo = jnp.zeros((16,), jnp.int32)
        run_hi = jnp.zeros((16,), jnp.int32)
        ranks = []
        for ev in e_sl[0] + e_sl[1]:
            sh_lo = jnp.minimum(8 * ev, 24)
            sh_hi = 8 * jnp.clip(ev - 4, 0, 3)
            enc_lo = jnp.where(ev < 4, jnp.left_shift(1, sh_lo), 0)
            enc_hi = jnp.where(ev >= 4, jnp.left_shift(1, sh_hi), 0)
            pe_lo = prefix_excl(enc_lo)
            pe_hi = prefix_excl(enc_hi)
            r = jnp.where(
                ev < 4,
                jnp.right_shift(run_lo + pe_lo, sh_lo) & 255,
                jnp.right_shift(run_hi + pe_hi, sh_hi) & 255)
            ranks.append(r)
            run_lo = run_lo + _gat(pe_lo + enc_lo, full15)
            run_hi = run_hi + _gat(pe_hi + enc_hi, full15)

        cvec = jnp.zeros((16,), jnp.int32)
        for e in range(_E):
            run = run_lo if e < 4 else run_hi
            cvec = cvec + jnp.where(
                iota == e, jnp.right_shift(run, 8 * (e % 4)) & 255, 0)
        cnt_v[...] = cvec
        pltpu.sync_copy(cnt_v, cnt_sh.at[pl.ds(sid * 16, 16)])
        plsc.subcore_barrier()
        pltpu.sync_copy(cnt_sh, allcnt_v)

        prior = jnp.zeros((16,), jnp.int32)
        total = jnp.zeros((16,), jnp.int32)
        sidv = iota * 0 + sid
        for t in range(_NTILE):
            ct = allcnt_v[pl.ds(t * 16, 16)]
            total = total + ct
            prior = prior + ct * jnp.clip(sidv - t, 0, 1)
        padded = ((total + (_TM - 1)) >> 8) << 8
        seg = prefix_excl(padded)
        base = seg + prior

        for k, ev in enumerate(e_sl[0] + e_sl[1]):
            pos = ranks[k] + _gat(base, ev)
            slot, v = divmod(k, ngroups)
            idx_v[slot, pl.ds(v * 16, 16)] = pos
            tokval_v[slot, pl.ds(v * 16, 16)] = t0 + v * 16 + iota
            wval_v[slot, pl.ds(v * 16, 16)] = w_sl[slot][v]

        for j in range(2):
            pltpu.sync_copy(tokval_v.at[j], src_tok_hbm.at[idx_v.at[j]])
            pltpu.sync_copy(wval_v.at[j], wsort_hbm.at[idx_v.at[j]])
        pltpu.sync_copy(idx_v.at[0], pos0_hbm.at[pl.ds(t0, _TOKT)])
        pltpu.sync_copy(idx_v.at[1], pos1_hbm.at[pl.ds(t0, _TOKT)])

        @pl.when(sid == 0)
        def _tiles():
            seg7 = _gat(seg, jnp.full((16,), 7, jnp.int32))
            pad7 = _gat(padded, jnp.full((16,), 7, jnp.int32))
            tot_pad = seg7 + pad7
            for half in range(2):
                mv = (iota + half * 16) * _TM
                te = jnp.zeros((16,), jnp.int32)
                for e in range(_E):
                    se = _gat(seg, jnp.full((16,), e, jnp.int32))
                    pe = _gat(padded, jnp.full((16,), e, jnp.int32))
                    inseg = jnp.logical_and(mv >= se, mv < se + pe)
                    te = te + jnp.where(inseg, e, 0)
                te = jnp.where(mv >= tot_pad, _E - 1, te)
                te_v[pl.ds(half * 16, 16)] = te
            pltpu.sync_copy(te_v, tile_e_hbm)


def _routing(gates):
    mesh = plsc.VectorSubcoreMesh(core_axis_name="c", subcore_axis_name="s")
    k = functools.partial(
        pl.kernel, mesh=mesh,
        out_type=[
            jax.ShapeDtypeStruct((_NPAD,), jnp.int32),
            jax.ShapeDtypeStruct((_NPAD,), jnp.float32),
            jax.ShapeDtypeStruct((32,), jnp.int32),
            jax.ShapeDtypeStruct((_S,), jnp.int32),
            jax.ShapeDtypeStruct((_S,), jnp.int32),
        ],
        scratch_types=[
            pltpu.VMEM((_E, _TOKT), jnp.float32),
            pltpu.VMEM((2, _TOKT), jnp.int32),
            pltpu.VMEM((2, _TOKT), jnp.int32),
            pltpu.VMEM((2, _TOKT), jnp.float32),
            pltpu.VMEM((16,), jnp.int32),
            pltpu.VMEM((_NTILE * 16,), jnp.int32),
            pltpu.VMEM((_NPAD // _NTILE,), jnp.int32),
            pltpu.VMEM((_NPAD // _NTILE,), jnp.float32),
            pltpu.VMEM((32,), jnp.int32),
            pltpu.VMEM_SHARED((_NTILE * 16,), jnp.int32),
        ],
    )(_route_body)
    return k(gates)


def _ffn_body(te_ref, st_ref, x_ref, w1_ref, b1_ref, w2_ref, b2_ref, ws_ref,
              y_ref, xbf_scr, xsort_scr):
    f = pl.program_id(0)
    m = pl.program_id(1)
    sl = pl.ds(m * _TM, _TM)

    @pl.when(jnp.logical_and(f == 0, m == 0))
    def _():
        xbf_scr[...] = x_ref[...].astype(jnp.bfloat16)

    @pl.when(f == 0)
    def _():
        tokm = st_ref[sl, :]
        oh = (jax.lax.broadcasted_iota(jnp.int32, (_TM, _S), 1)
              == tokm).astype(jnp.bfloat16)
        xm_s = jax.lax.dot_general(
            oh, xbf_scr[...], (((1,), (0,)), ((), ())),
            preferred_element_type=jnp.float32)
        xsort_scr[sl, :] = xm_s.astype(jnp.bfloat16)

    xm = xsort_scr[sl, :]
    h = jax.lax.dot_general(
        xm, w1_ref[0].astype(jnp.bfloat16),
        (((1,), (0,)), ((), ())), preferred_element_type=jnp.float32)
    h = h + b1_ref[0]
    h = 0.5 * h * (1.0 + jax.lax.erf(h * 0.7071067811865476))
    part = jax.lax.dot_general(
        h.astype(jnp.bfloat16), w2_ref[0].astype(jnp.bfloat16),
        (((1,), (0,)), ((), ())), preferred_element_type=jnp.float32)

    @pl.when(f == 0)
    def _():
        y_ref[sl, :] = part + b2_ref[0]

    @pl.when(jnp.logical_and(f > 0, f < _NF - 1))
    def _():
        y_ref[sl, :] += part

    @pl.when(f == _NF - 1)
    def _():
        y_ref[sl, :] = (y_ref[sl, :] + part) * ws_ref[sl, :]


def _ffn(tile_e, src_tok, xs, W1, b1, W2, b2, wsort):
    grid_spec = pltpu.PrefetchScalarGridSpec(
        num_scalar_prefetch=1,
        grid=(_NF, _GM),
        in_specs=[
            pl.BlockSpec((_NPAD, 1), lambda f, m, te: (0, 0)),
            pl.BlockSpec((_S, _D), lambda f, m, te: (0, 0)),
            pl.BlockSpec((1, _D, _FT), lambda f, m, te: (te[m], 0, f)),
            pl.BlockSpec((1, 1, _FT), lambda f, m, te: (te[m], 0, f)),
            pl.BlockSpec((1, _FT, _D), lambda f, m, te: (te[m], f, 0)),
            pl.BlockSpec((1, 1, _D), lambda f, m, te: (te[m], 0, 0)),
            pl.BlockSpec((_NPAD, 1), lambda f, m, te: (0, 0)),
        ],
        out_specs=pl.BlockSpec((_NPAD, _D), lambda f, m, te: (0, 0)),
        scratch_shapes=[pltpu.VMEM((_S, _D), jnp.bfloat16),
                        pltpu.VMEM((_NPAD, _D), jnp.bfloat16)],
    )
    return pl.pallas_call(
        _ffn_body,
        grid_spec=grid_spec,
        out_shape=jax.ShapeDtypeStruct((_NPAD, _D), jnp.float32),
    )(tile_e, src_tok.reshape(_NPAD, 1), xs, W1, b1.reshape(_E, 1, _H), W2,
      b2.reshape(_E, 1, _D), wsort.reshape(_NPAD, 1))


def _combine_body(pos0_hbm, pos1_hbm, y_hbm, out_hbm,
                  p0_v, p1_v, ra_v, rb_v, sem):
    wid = lax.axis_index("s") * 2 + lax.axis_index("c")
    toks = _S // 32
    ck = 16
    t0 = wid * toks
    pltpu.sync_copy(pos0_hbm.at[pl.ds(t0, toks)], p0_v)
    pltpu.sync_copy(pos1_hbm.at[pl.ds(t0, toks)], p1_v)

    def chunk(j, carry):
        pltpu.async_copy(y_hbm.at[p0_v.at[pl.ds(j * ck, ck)]], ra_v,
                         sem).wait()
        pltpu.async_copy(y_hbm.at[p1_v.at[pl.ds(j * ck, ck)]], rb_v,
                         sem).wait()
        for r in range(ck):
            for c in range(_D // 16):
                cs = pl.ds(c * 16, 16)
                rb_v[r, cs] = ra_v[r, cs] + rb_v[r, cs]
        pltpu.sync_copy(rb_v, out_hbm.at[pl.ds(t0 + j * ck, ck)])
        return carry

    lax.fori_loop(0, toks // ck, chunk, 0)


def _combine(pos0, pos1, y_sorted):
    mesh = plsc.VectorSubcoreMesh(core_axis_name="c", subcore_axis_name="s")
    k = functools.partial(
        pl.kernel, mesh=mesh,
        out_type=jax.ShapeDtypeStruct((_S, _D), jnp.float32),
        scratch_types=[
            pltpu.VMEM((_S // 32,), jnp.int32),
            pltpu.VMEM((_S // 32,), jnp.int32),
            pltpu.VMEM((16, _D), jnp.float32),
            pltpu.VMEM((16, _D), jnp.float32),
            pltpu.SemaphoreType.DMA,
        ],
    )(_combine_body)
    return k(pos0, pos1, y_sorted)


def kernel(x, Wg, bg, W1, b1, W2, b2):
    xs = x.reshape(_S, _D)
    gates = _gating(xs, Wg, bg)
    src_tok, wsort, tile_e, pos0, pos1 = _routing(gates)
    y_sorted = _ffn(tile_e, src_tok, xs, W1, b1, W2, b2, wsort)
    out = _combine(pos0, pos1, y_sorted)
    return out.reshape(1, _S, _D)


# speedup vs baseline: 1.7732x; 1.0115x over previous
import functools

import jax
import jax.numpy as jnp
from jax import lax
from jax.experimental import pallas as pl
from jax.experimental.pallas import tpu as pltpu
from jax.experimental.pallas import tpu_sc as plsc

_D = 768
_E = 8
_S = 2048
_H = 3072
_FT = 768
_NF = _H // _FT
_TM = 256
_NPAD = 4096 + _E * _TM
_GM = _NPAD // _TM
_NEG = -1e30
_NTILE = 16
_TOKT = _S // _NTILE


def _gate_body(x_ref, wg_ref, bg_ref, g_ref):
    xb = x_ref[...].astype(jnp.bfloat16)
    g = jax.lax.dot_general(
        xb, wg_ref[...].astype(jnp.bfloat16),
        (((1,), (0,)), ((), ())), preferred_element_type=jnp.float32)
    g = g + bg_ref[...]
    g_ref[...] = jnp.transpose(g[:, :_E])


def _gating(xs, Wg, bg):
    wg_pad = jnp.pad(Wg, ((0, 0), (0, 128 - _E)))
    bg_pad = jnp.pad(bg, (0, 128 - _E)).reshape(1, 128)
    return pl.pallas_call(
        _gate_body,
        grid=(_S // _TM,),
        in_specs=[
            pl.BlockSpec((_TM, _D), lambda m: (m, 0)),
            pl.BlockSpec((_D, 128), lambda m: (0, 0)),
            pl.BlockSpec((1, 128), lambda m: (0, 0)),
        ],
        out_specs=pl.BlockSpec((_E, _TM), lambda m: (0, m)),
        out_shape=jax.ShapeDtypeStruct((_E, _S), jnp.float32),
    )(xs, wg_pad, bg_pad)


def _gat(x, idx):
    return x.at[idx].get(mode='promise_in_bounds')


def _route_body(g_hbm, src_tok_hbm, wsort_hbm, tile_e_hbm, pos0_hbm, pos1_hbm,
                g_v, idx_v, tokval_v, wval_v, cnt_v, allcnt_v, zi_v, zf_v,
                te_v, cnt_sh, semg, semo):
    cid = lax.axis_index("c")
    sid = lax.axis_index("s")

    @pl.when(cid == 0)
    def _():
        t0 = sid * _TOKT
        gh = [pltpu.async_copy(g_hbm.at[e, pl.ds(t0, _TOKT)], g_v.at[e],
                               semg) for e in range(_E)]

        zslice = _NPAD // _NTILE
        z16i = jnp.zeros((16,), jnp.int32)
        z16f = jnp.zeros((16,), jnp.float32)
        for i in range(zslice // 16):
            zi_v[pl.ds(i * 16, 16)] = z16i
            zf_v[pl.ds(i * 16, 16)] = z16f
        pltpu.sync_copy(zi_v, src_tok_hbm.at[pl.ds(sid * zslice, zslice)])
        pltpu.sync_copy(zf_v, wsort_hbm.at[pl.ds(sid * zslice, zslice)])
        for h in gh:
            h.wait()

        iota = lax.iota(jnp.int32, 16)

        def shdown(x, k):
            g = _gat(x, jnp.maximum(iota - k, 0))
            return jnp.where(iota >= k, g, 0)

        def prefix_excl(x):
            s = x
            for k in (1, 2, 4, 8):
                s = s + shdown(s, k)
            return s - x

        ngroups = _TOKT // 16
        e_sl, w_sl = [[], []], [[], []]
        for v in range(ngroups):
            gs = [g_v[e, pl.ds(v * 16, 16)] for e in range(_E)]
            m1 = gs[0]
            i1 = jnp.zeros((16,), jnp.int32)
            for e in range(1, _E):
                better = gs[e] > m1
                m1 = jnp.where(better, gs[e], m1)
                i1 = jnp.where(better, e, i1)
            m2 = jnp.full((16,), _NEG, jnp.float32)
            i2 = jnp.zeros((16,), jnp.int32)
            for e in range(_E):
                cand = jnp.where(i1 == e, _NEG, gs[e])
                better = cand > m2
                m2 = jnp.where(better, cand, m2)
                i2 = jnp.where(better, e, i2)
            p1 = 1.0 / (1.0 + jnp.exp(m2 - m1))
            e_sl[0].append(i1)
            e_sl[1].append(i2)
            w_sl[0].append(p1)
            w_sl[1].append(1.0 - p1)

        full15 = jnp.full((16,), 15, jnp.int32)
        run_lo = jnp.zeros((16,), jnp.int32)
        run_hi = jnp.zeros((16,), jnp.int32)
        ranks = []
        for ev in e_sl[0] + e_sl[1]:
            sh_lo = jnp.minimum(8 * ev, 24)
            sh_hi = 8 * jnp.clip(ev - 4, 0, 3)
            enc_lo = jnp.where(ev < 4, jnp.left_shift(1, sh_lo), 0)
            enc_hi = jnp.where(ev >= 4, jnp.left_shift(1, sh_hi), 0)
            pe_lo = prefix_excl(enc_lo)
            pe_hi = prefix_excl(enc_hi)
            r = jnp.where(
                ev < 4,
                jnp.right_shift(run_lo + pe_lo, sh_lo) & 255,
                jnp.right_shift(run_hi + pe_hi, sh_hi) & 255)
            ranks.append(r)
            run_lo = run_lo + _gat(pe_lo + enc_lo, full15)
            run_hi = run_hi + _gat(pe_hi + enc_hi, full15)

        cvec = jnp.zeros((16,), jnp.int32)
        for e in range(_E):
            run = run_lo if e < 4 else run_hi
            cvec = cvec + jnp.where(
                iota == e, jnp.right_shift(run, 8 * (e % 4)) & 255, 0)
        cnt_v[...] = cvec
        pltpu.sync_copy(cnt_v, cnt_sh.at[pl.ds(sid * 16, 16)])
        plsc.subcore_barrier()
        pltpu.sync_copy(cnt_sh, allcnt_v)

        prior = jnp.zeros((16,), jnp.int32)
        total = jnp.zeros((16,), jnp.int32)
        sidv = iota * 0 + sid
        for t in range(_NTILE):
            ct = allcnt_v[pl.ds(t * 16, 16)]
            total = total + ct
            prior = prior + ct * jnp.clip(sidv - t, 0, 1)
        padded = ((total + (_TM - 1)) >> 8) << 8
        seg = prefix_excl(padded)
        base = seg + prior

        for k, ev in enumerate(e_sl[0] + e_sl[1]):
            pos = ranks[k] + _gat(base, ev)
            slot, v = divmod(k, ngroups)
            idx_v[slot, pl.ds(v * 16, 16)] = pos
            tokval_v[slot, pl.ds(v * 16, 16)] = t0 + v * 16 + iota
            wval_v[slot, pl.ds(v * 16, 16)] = w_sl[slot][v]

        oh = []
        for j in range(2):
            oh.append(pltpu.async_copy(tokval_v.at[j],
                                       src_tok_hbm.at[idx_v.at[j]], semo))
            oh.append(pltpu.async_copy(wval_v.at[j],
                                       wsort_hbm.at[idx_v.at[j]], semo))
        oh.append(pltpu.async_copy(idx_v.at[0],
                                   pos0_hbm.at[pl.ds(t0, _TOKT)], semo))
        oh.append(pltpu.async_copy(idx_v.at[1],
                                   pos1_hbm.at[pl.ds(t0, _TOKT)], semo))
        for h in oh:
            h.wait()

        @pl.when(sid == 0)
        def _tiles():
            seg7 = _gat(seg, jnp.full((16,), 7, jnp.int32))
            pad7 = _gat(padded, jnp.full((16,), 7, jnp.int32))
            tot_pad = seg7 + pad7
            for half in range(2):
                mv = (iota + half * 16) * _TM
                te = jnp.zeros((16,), jnp.int32)
                for e in range(_E):
                    se = _gat(seg, jnp.full((16,), e, jnp.int32))
                    pe = _gat(padded, jnp.full((16,), e, jnp.int32))
                    inseg = jnp.logical_and(mv >= se, mv < se + pe)
                    te = te + jnp.where(inseg, e, 0)
                te = jnp.where(mv >= tot_pad, _E - 1, te)
                te_v[pl.ds(half * 16, 16)] = te
            pltpu.sync_copy(te_v, tile_e_hbm)


def _routing(gates):
    mesh = plsc.VectorSubcoreMesh(core_axis_name="c", subcore_axis_name="s")
    k = functools.partial(
        pl.kernel, mesh=mesh,
        out_type=[
            jax.ShapeDtypeStruct((_NPAD,), jnp.int32),
            jax.ShapeDtypeStruct((_NPAD,), jnp.float32),
            jax.ShapeDtypeStruct((32,), jnp.int32),
            jax.ShapeDtypeStruct((_S,), jnp.int32),
            jax.ShapeDtypeStruct((_S,), jnp.int32),
        ],
        scratch_types=[
            pltpu.VMEM((_E, _TOKT), jnp.float32),
            pltpu.VMEM((2, _TOKT), jnp.int32),
            pltpu.VMEM((2, _TOKT), jnp.int32),
            pltpu.VMEM((2, _TOKT), jnp.float32),
            pltpu.VMEM((16,), jnp.int32),
            pltpu.VMEM((_NTILE * 16,), jnp.int32),
            pltpu.VMEM((_NPAD // _NTILE,), jnp.int32),
            pltpu.VMEM((_NPAD // _NTILE,), jnp.float32),
            pltpu.VMEM((32,), jnp.int32),
            pltpu.VMEM_SHARED((_NTILE * 16,), jnp.int32),
            pltpu.SemaphoreType.DMA,
            pltpu.SemaphoreType.DMA,
        ],
    )(_route_body)
    return k(gates)


def _ffn_body(te_ref, st_ref, x_ref, w1_ref, b1_ref, w2_ref, b2_ref, ws_ref,
              y_ref, xbf_scr, xsort_scr):
    f = pl.program_id(0)
    m = pl.program_id(1)
    sl = pl.ds(m * _TM, _TM)

    @pl.when(jnp.logical_and(f == 0, m == 0))
    def _():
        xbf_scr[...] = x_ref[...].astype(jnp.bfloat16)

    @pl.when(f == 0)
    def _():
        tokm = st_ref[sl, :]
        oh = (jax.lax.broadcasted_iota(jnp.int32, (_TM, _S), 1)
              == tokm).astype(jnp.bfloat16)
        xm_s = jax.lax.dot_general(
            oh, xbf_scr[...], (((1,), (0,)), ((), ())),
            preferred_element_type=jnp.float32)
        xsort_scr[sl, :] = xm_s.astype(jnp.bfloat16)

    xm = xsort_scr[sl, :]
    h = jax.lax.dot_general(
        xm, w1_ref[0].astype(jnp.bfloat16),
        (((1,), (0,)), ((), ())), preferred_element_type=jnp.float32)
    h = h + b1_ref[0]
    h = 0.5 * h * (1.0 + jax.lax.erf(h * 0.7071067811865476))
    part = jax.lax.dot_general(
        h.astype(jnp.bfloat16), w2_ref[0].astype(jnp.bfloat16),
        (((1,), (0,)), ((), ())), preferred_element_type=jnp.float32)

    @pl.when(f == 0)
    def _():
        y_ref[sl, :] = part + b2_ref[0]

    @pl.when(jnp.logical_and(f > 0, f < _NF - 1))
    def _():
        y_ref[sl, :] += part

    @pl.when(f == _NF - 1)
    def _():
        y_ref[sl, :] = (y_ref[sl, :] + part) * ws_ref[sl, :]


def _ffn(tile_e, src_tok, xs, W1, b1, W2, b2, wsort):
    grid_spec = pltpu.PrefetchScalarGridSpec(
        num_scalar_prefetch=1,
        grid=(_NF, _GM),
        in_specs=[
            pl.BlockSpec((_NPAD, 1), lambda f, m, te: (0, 0)),
            pl.BlockSpec((_S, _D), lambda f, m, te: (0, 0)),
            pl.BlockSpec((1, _D, _FT), lambda f, m, te: (te[m], 0, f)),
            pl.BlockSpec((1, 1, _FT), lambda f, m, te: (te[m], 0, f)),
            pl.BlockSpec((1, _FT, _D), lambda f, m, te: (te[m], f, 0)),
            pl.BlockSpec((1, 1, _D), lambda f, m, te: (te[m], 0, 0)),
            pl.BlockSpec((_NPAD, 1), lambda f, m, te: (0, 0)),
        ],
        out_specs=pl.BlockSpec((_NPAD, _D), lambda f, m, te: (0, 0)),
        scratch_shapes=[pltpu.VMEM((_S, _D), jnp.bfloat16),
                        pltpu.VMEM((_NPAD, _D), jnp.bfloat16)],
    )
    return pl.pallas_call(
        _ffn_body,
        grid_spec=grid_spec,
        out_shape=jax.ShapeDtypeStruct((_NPAD, _D), jnp.float32),
    )(tile_e, src_tok.reshape(_NPAD, 1), xs, W1, b1.reshape(_E, 1, _H), W2,
      b2.reshape(_E, 1, _D), wsort.reshape(_NPAD, 1))


def _combine_body(pos0_hbm, pos1_hbm, y_hbm, out_hbm,
                  p0_v, p1_v, ra_v, rb_v, sem):
    wid = lax.axis_index("s") * 2 + lax.axis_index("c")
    toks = _S // 32
    ck = 16
    t0 = wid * toks
    pltpu.sync_copy(pos0_hbm.at[pl.ds(t0, toks)], p0_v)
    pltpu.sync_copy(pos1_hbm.at[pl.ds(t0, toks)], p1_v)

    def chunk(j, carry):
        pltpu.async_copy(y_hbm.at[p0_v.at[pl.ds(j * ck, ck)]], ra_v,
                         sem).wait()
        pltpu.async_copy(y_hbm.at[p1_v.at[pl.ds(j * ck, ck)]], rb_v,
                         sem).wait()
        for r in range(ck):
            for c in range(_D // 16):
                cs = pl.ds(c * 16, 16)
                rb_v[r, cs] = ra_v[r, cs] + rb_v[r, cs]
        pltpu.sync_copy(rb_v, out_hbm.at[pl.ds(t0 + j * ck, ck)])
        return carry

    lax.fori_loop(0, toks // ck, chunk, 0)


def _combine(pos0, pos1, y_sorted):
    mesh = plsc.VectorSubcoreMesh(core_axis_name="c", subcore_axis_name="s")
    k = functools.partial(
        pl.kernel, mesh=mesh,
        out_type=jax.ShapeDtypeStruct((_S, _D), jnp.float32),
        scratch_types=[
            pltpu.VMEM((_S // 32,), jnp.int32),
            pltpu.VMEM((_S // 32,), jnp.int32),
            pltpu.VMEM((16, _D), jnp.float32),
            pltpu.VMEM((16, _D), jnp.float32),
            pltpu.SemaphoreType.DMA,
        ],
    )(_combine_body)
    return k(pos0, pos1, y_sorted)


def kernel(x, Wg, bg, W1, b1, W2, b2):
    xs = x.reshape(_S, _D)
    gates = _gating(xs, Wg, bg)
    src_tok, wsort, tile_e, pos0, pos1 = _routing(gates)
    y_sorted = _ffn(tile_e, src_tok, xs, W1, b1, W2, b2, wsort)
    out = _combine(pos0, pos1, y_sorted)
    return out.reshape(1, _S, _D)
